# baseline (speedup 1.0000x reference)
import math

import jax
import jax.numpy as jnp
from jax import lax
from jax.experimental import pallas as pl
from jax.experimental.pallas import tpu as pltpu
from jax.experimental.pallas import tpu_sc as plsc

D_MODEL = 64
D_PAD = 128
SCALE = math.sqrt(D_MODEL)
CHUNK = 128

_info = plsc.get_sparse_core_info()
NUM_CORES = _info.num_cores
NUM_SUBCORES = _info.num_subcores
NW = NUM_CORES * NUM_SUBCORES


def _make_tc_format(vocab: int):
    BVT = 8192
    grid = (vocab + BVT - 1) // BVT

    def body(tt_ref, out_ref):
        blk = tt_ref[...]
        t = jnp.transpose(blk, (1, 0)) * SCALE
        out_ref[...] = jnp.concatenate(
            [t, jnp.zeros((BVT, D_PAD - D_MODEL), jnp.float32)], axis=1)

    return pl.pallas_call(
        body,
        grid=(grid,),
        in_specs=[pl.BlockSpec((D_MODEL, BVT), lambda i: (0, i))],
        out_specs=pl.BlockSpec((BVT, D_PAD), lambda i: (i, 0)),
        out_shape=jax.ShapeDtypeStruct((vocab, D_PAD), jnp.float32),
    )


def _make_sc_lookup(n_chunks_total: int):
    n_chunks = n_chunks_total // NW
    assert n_chunks * NW == n_chunks_total and n_chunks % 2 == 0
    n_pairs = n_chunks // 2

    mesh = plsc.VectorSubcoreMesh(core_axis_name="c", subcore_axis_name="s")

    @pl.kernel(
        out_type=jax.ShapeDtypeStruct((n_chunks_total * CHUNK, D_MODEL),
                                      jnp.float32),
        mesh=mesh,
        scratch_types=[
            pltpu.VMEM((n_chunks, CHUNK), jnp.int32),
            pltpu.VMEM((CHUNK, D_PAD), jnp.float32),
            pltpu.VMEM((CHUNK, D_PAD), jnp.float32),
            pltpu.VMEM((CHUNK, D_MODEL), jnp.float32),
            pltpu.VMEM((CHUNK, D_MODEL), jnp.float32),
            pltpu.SemaphoreType.DMA,
            pltpu.SemaphoreType.DMA,
            pltpu.SemaphoreType.DMA,
            pltpu.SemaphoreType.DMA,
        ],
    )
    def lookup(idx_hbm, table_hbm, out_hbm,
               idx_v, g0, g1, s0, s1, gsem0, gsem1, ssem0, ssem1):
        wid = lax.axis_index("s") * NUM_CORES + lax.axis_index("c")

        pltpu.sync_copy(idx_hbm.at[pl.ds(wid * n_chunks, n_chunks)], idx_v)

        gbuf = (g0, g1)
        sbuf = (s0, s1)
        gsem = (gsem0, gsem1)
        ssem = (ssem0, ssem1)

        def gather_start(j, b):
            pltpu.make_async_copy(
                table_hbm.at[idx_v.at[j]], gbuf[b], gsem[b]).start()

        def scale(b):
            src, dst = gbuf[b], sbuf[b]

            def body(r, carry):
                base = r * 8
                for k in range(8):
                    for c in range(D_MODEL // 16):
                        v = src[base + k, pl.ds(c * 16, 16)]
                        dst[base + k, pl.ds(c * 16, 16)] = v
                return carry

            lax.fori_loop(0, CHUNK // 8, body, 0)

        def scatter(j, b):
            return pltpu.make_async_copy(
                sbuf[b],
                out_hbm.at[pl.ds((wid * n_chunks + j) * CHUNK, CHUNK)],
                ssem[b])

        gather_start(0, 0)
        gather_start(1, 1)

        def pair(i, carry):
            for b in range(2):
                j = i * 2 + b
                pltpu.make_async_copy(
                    table_hbm.at[idx_v.at[j]], gbuf[b], gsem[b]).wait()

                @pl.when(i >= 1)
                def _wait_prev_scatter():
                    scatter(j - 2, b).wait()

                scale(b)

                @pl.when(i < n_pairs - 1)
                def _next_gather():
                    gather_start(j + 2, b)

                scatter(j, b).start()
            return carry

        lax.fori_loop(0, n_pairs, pair, 0)

        for b in range(2):
            scatter(n_chunks - 2 + b, b).wait()

    return lookup


def kernel(x, table):
    b, s = x.shape
    n = b * s
    assert n % (NW * CHUNK * 2) == 0, (b, s)
    idx = x.reshape(n // CHUNK, CHUNK).astype(jnp.int32)
    vocab = table.shape[0]
    tablep = _make_tc_format(vocab)(table.T)
    lookup = _make_sc_lookup(n // CHUNK)
    out = lookup(idx, tablep)
    return out.reshape(b, s, D_MODEL)
